# baseline (speedup 1.0000x reference)
import functools

import jax
import jax.numpy as jnp
from jax import lax
from jax.experimental import pallas as pl
from jax.experimental.pallas import tpu as pltpu
from jax.experimental.pallas import tpu_sc as plsc

B = 16
MAX_LEN = 2048
D = 1024

NC = 2
NS = 16
NW = NC * NS
ROWS_PER_W = B * MAX_LEN // NW
CHUNK = 64
NCHUNK = ROWS_PER_W // CHUNK
HALF = 32

_mesh = plsc.VectorSubcoreMesh(core_axis_name="c", subcore_axis_name="s")


@functools.partial(
    pl.kernel,
    mesh=_mesh,
    out_type=jax.ShapeDtypeStruct((B * MAX_LEN, D), jnp.float32),
    scratch_types=[
        pltpu.VMEM((16,), jnp.int32),
        pltpu.VMEM((HALF,), jnp.int32),
        pltpu.VMEM((HALF,), jnp.int32),
        pltpu.VMEM((HALF, D), jnp.float32),
        pltpu.VMEM((HALF, D), jnp.float32),
        pltpu.VMEM((HALF, D), jnp.float32),
        pltpu.SemaphoreType.DMA,
        pltpu.SemaphoreType.DMA,
    ],
)
def _pos_enc_sc(len_hbm, table_hbm, tshift_hbm, out_hbm, len_v, idxz_v,
                idx_v, pad_v, buf_a, buf_b, gsem, csem):
    wid = lax.axis_index("s") * NC + lax.axis_index("c")
    b = wid // 2
    half = wid % 2
    i0 = half * ROWS_PER_W
    row_out0 = wid * ROWS_PER_W

    lane = lax.iota(jnp.int32, 16)
    zeros = jnp.zeros((16,), jnp.int32)
    for j in range(HALF // 16):
        idxz_v[pl.ds(j * 16, 16)] = zeros

    pltpu.sync_copy(len_hbm.at[pl.ds(wid * 16, 16)], len_v)
    len_b = len_v[...][0]

    pltpu.async_copy(table_hbm.at[idxz_v], pad_v, gsem).wait()

    bufs = (buf_a, buf_b)

    def chunk_body(c, _):
        base_i = i0 + c * CHUNK
        row_out = row_out0 + c * CHUNK
        is_data = base_i + CHUNK <= len_b
        is_pad = base_i >= len_b

        @pl.when(is_data)
        def _():
            pltpu.sync_copy(tshift_hbm.at[pl.ds(base_i, CHUNK)],
                            out_hbm.at[pl.ds(row_out, CHUNK)])

        @pl.when(jnp.logical_and(~is_data, is_pad))
        def _():
            for h in range(CHUNK // HALF):
                pltpu.sync_copy(pad_v,
                                out_hbm.at[pl.ds(row_out + h * HALF, HALF)])

        @pl.when(jnp.logical_and(~is_data, ~is_pad))
        def _():
            for h in range(CHUNK // HALF):
                for j in range(HALF // 16):
                    rows = base_i + h * HALF + j * 16 + lane
                    idx_v[pl.ds(j * 16, 16)] = jnp.where(
                        rows < len_b, rows + 1, 0)
                pltpu.async_copy(table_hbm.at[idx_v], bufs[h], gsem).wait()
                pltpu.sync_copy(bufs[h],
                                out_hbm.at[pl.ds(row_out + h * HALF, HALF)])

        return _

    lax.fori_loop(0, NCHUNK, chunk_body, None)


def kernel(input_len, table):
    tshift = lax.slice(table, (1, 0), (MAX_LEN + 1, D))
    len_rep = jnp.repeat(input_len, 2 * 16)
    out = _pos_enc_sc(len_rep, table, tshift)
    return out.reshape(B, MAX_LEN, D)


# speedup vs baseline: 1.0006x; 1.0006x over previous
import functools

import jax
import jax.numpy as jnp
from jax import lax
from jax.experimental import pallas as pl
from jax.experimental.pallas import tpu as pltpu
from jax.experimental.pallas import tpu_sc as plsc

B = 16
MAX_LEN = 2048
D = 1024

NC = 2
NS = 16
NW = NC * NS
ROWS_PER_W = B * MAX_LEN // NW
CHUNK = 64
NCHUNK = ROWS_PER_W // CHUNK
HALF = 32

_mesh = plsc.VectorSubcoreMesh(core_axis_name="c", subcore_axis_name="s")


@functools.partial(
    pl.kernel,
    mesh=_mesh,
    out_type=jax.ShapeDtypeStruct((B * MAX_LEN, D), jnp.float32),
    scratch_types=[
        pltpu.VMEM((16,), jnp.int32),
        pltpu.VMEM((HALF,), jnp.int32),
        pltpu.VMEM((HALF,), jnp.int32),
        pltpu.VMEM((HALF, D), jnp.float32),
        pltpu.VMEM((HALF, D), jnp.float32),
        pltpu.VMEM((HALF, D), jnp.float32),
        pltpu.SemaphoreType.DMA,
        pltpu.SemaphoreType.DMA,
    ],
)
def _pos_enc_sc(len_hbm, table_hbm, tshift_hbm, out_hbm, len_v, idxz_v,
                idx_v, pad_v, buf_a, buf_b, gsem, csem):
    wid = lax.axis_index("s") * NC + lax.axis_index("c")
    b = wid // 2
    half = wid % 2
    i0 = half * ROWS_PER_W
    row_out0 = wid * ROWS_PER_W

    lane = lax.iota(jnp.int32, 16)
    zeros = jnp.zeros((16,), jnp.int32)
    for j in range(HALF // 16):
        idxz_v[pl.ds(j * 16, 16)] = zeros

    pltpu.sync_copy(len_hbm.at[pl.ds(wid * 16, 16)], len_v)
    len_b = len_v[...][0]

    pltpu.async_copy(table_hbm.at[idxz_v], pad_v, gsem).wait()

    bufs = (buf_a, buf_b)

    def chunk_body(c, _):
        base_i = i0 + c * CHUNK
        row_out = row_out0 + c * CHUNK
        is_data = base_i + CHUNK <= len_b
        is_pad = base_i >= len_b

        @pl.when(is_data)
        def _():
            pltpu.async_copy(tshift_hbm.at[pl.ds(base_i, CHUNK)],
                             out_hbm.at[pl.ds(row_out, CHUNK)], csem)

        @pl.when(jnp.logical_and(~is_data, is_pad))
        def _():
            for h in range(CHUNK // HALF):
                pltpu.async_copy(pad_v,
                                 out_hbm.at[pl.ds(row_out + h * HALF, HALF)],
                                 csem)

        @pl.when(jnp.logical_and(~is_data, ~is_pad))
        def _():
            for h in range(CHUNK // HALF):
                for j in range(HALF // 16):
                    rows = base_i + h * HALF + j * 16 + lane
                    idx_v[pl.ds(j * 16, 16)] = jnp.where(
                        rows < len_b, rows + 1, 0)
                pltpu.async_copy(table_hbm.at[idx_v], bufs[h], gsem).wait()
                pltpu.async_copy(bufs[h],
                                 out_hbm.at[pl.ds(row_out + h * HALF, HALF)],
                                 csem)

        return _

    lax.fori_loop(0, NCHUNK, chunk_body, None)

    def drain_body(c, _):
        base_i = i0 + c * CHUNK
        row_out = row_out0 + c * CHUNK
        is_data = base_i + CHUNK <= len_b
        is_pad = base_i >= len_b

        @pl.when(is_data)
        def _():
            pltpu.make_async_copy(tshift_hbm.at[pl.ds(base_i, CHUNK)],
                                  out_hbm.at[pl.ds(row_out, CHUNK)],
                                  csem).wait()

        @pl.when(jnp.logical_and(~is_data, is_pad))
        def _():
            for h in range(CHUNK // HALF):
                pltpu.make_async_copy(
                    pad_v, out_hbm.at[pl.ds(row_out + h * HALF, HALF)],
                    csem).wait()

        @pl.when(jnp.logical_and(~is_data, ~is_pad))
        def _():
            for h in range(CHUNK // HALF):
                pltpu.make_async_copy(
                    bufs[h], out_hbm.at[pl.ds(row_out + h * HALF, HALF)],
                    csem).wait()

        return _

    lax.fori_loop(0, NCHUNK, drain_body, None)


def kernel(input_len, table):
    tshift = lax.slice(table, (1, 0), (MAX_LEN + 1, D))
    len_rep = jnp.repeat(input_len, 2 * 16)
    out = _pos_enc_sc(len_rep, table, tshift)
    return out.reshape(B, MAX_LEN, D)


# speedup vs baseline: 2.4020x; 2.4006x over previous
import functools

import jax
import jax.numpy as jnp
from jax import lax
from jax.experimental import pallas as pl
from jax.experimental.pallas import tpu as pltpu
from jax.experimental.pallas import tpu_sc as plsc

B = 16
MAX_LEN = 2048
D = 1024

NC = 2
NS = 16
NW = NC * NS
ROWS_PER_W = B * MAX_LEN // NW
CHUNK = 32
NCHUNK = ROWS_PER_W // CHUNK
NBUF = 3
NGROUP = NCHUNK // NBUF
NTAIL = NCHUNK - NGROUP * NBUF

_mesh = plsc.VectorSubcoreMesh(core_axis_name="c", subcore_axis_name="s")


@functools.partial(
    pl.kernel,
    mesh=_mesh,
    out_type=jax.ShapeDtypeStruct((B * MAX_LEN, D), jnp.float32),
    scratch_types=[
        pltpu.VMEM((16,), jnp.int32),
        pltpu.VMEM((NBUF, CHUNK), jnp.int32),
        pltpu.VMEM((NBUF, CHUNK, D), jnp.float32),
        pltpu.SemaphoreType.DMA((NBUF,)),
        pltpu.SemaphoreType.DMA((NBUF,)),
    ],
)
def _pos_enc_sc(len_hbm, table_hbm, out_hbm, len_v, idx_v, buf_v, gsem, ssem):
    wid = lax.axis_index("s") * NC + lax.axis_index("c")
    half = wid % 2
    i0 = half * ROWS_PER_W
    row_out0 = wid * ROWS_PER_W

    lane = lax.iota(jnp.int32, 16)

    pltpu.sync_copy(len_hbm.at[pl.ds(wid * 16, 16)], len_v)
    len_b = len_v[...][0]

    def fire_gather(s, c):
        base_i = i0 + c * CHUNK
        for j in range(CHUNK // 16):
            rows = base_i + j * 16 + lane
            idx_v.at[s][pl.ds(j * 16, 16)] = jnp.where(rows < len_b, rows + 1,
                                                       0)
        pltpu.async_copy(table_hbm.at[idx_v.at[s]], buf_v.at[s], gsem.at[s])

    def wait_gather(s):
        pltpu.make_async_copy(table_hbm.at[idx_v.at[s]], buf_v.at[s],
                              gsem.at[s]).wait()

    def fire_scatter(s, c):
        pltpu.async_copy(buf_v.at[s],
                         out_hbm.at[pl.ds(row_out0 + c * CHUNK, CHUNK)],
                         ssem.at[s])

    def wait_scatter(s, c):
        pltpu.make_async_copy(buf_v.at[s],
                              out_hbm.at[pl.ds(row_out0 + c * CHUNK, CHUNK)],
                              ssem.at[s]).wait()

    def step(c, s, guard):
        sp = (s + NBUF - 1) % NBUF
        if guard:
            @pl.when(c >= NBUF)
            def _():
                wait_scatter(s, c - NBUF)
        else:
            wait_scatter(s, c - NBUF)
        fire_gather(s, c)
        if guard:
            @pl.when(c >= 1)
            def _():
                wait_gather(sp)
                fire_scatter(sp, c - 1)
        else:
            wait_gather(sp)
            fire_scatter(sp, c - 1)

    def group_body(g, _):
        for s in range(NBUF):
            step(g * NBUF + s, s, guard=True)
        return _

    lax.fori_loop(0, NGROUP, group_body, None)

    for t in range(NTAIL):
        step(NGROUP * NBUF + t, t, guard=False)

    last = NCHUNK - 1
    wait_gather(last % NBUF)
    fire_scatter(last % NBUF, last)
    for c in range(NCHUNK - NBUF, NCHUNK):
        wait_scatter(c % NBUF, c)


def kernel(input_len, table):
    len_rep = jnp.repeat(input_len, 2 * 16)
    out = _pos_enc_sc(len_rep, table)
    return out.reshape(B, MAX_LEN, D)


# speedup vs baseline: 17.4259x; 7.2546x over previous
import functools

import jax
import jax.numpy as jnp
from jax import lax
from jax.experimental import pallas as pl
from jax.experimental.pallas import tpu as pltpu
from jax.experimental.pallas import tpu_sc as plsc

B = 16
MAX_LEN = 2048
D = 1024

NC = 2
NS = 16
HALF_LEN = MAX_LEN // NC
CHUNK = 64
NCHUNK = HALF_LEN // CHUNK
HB = 32
NPHASE = 2
SPROWS = HALF_LEN // NPHASE
PCHUNK = NCHUNK // NPHASE

_mesh = plsc.VectorSubcoreMesh(core_axis_name="c", subcore_axis_name="s")


@functools.partial(
    pl.kernel,
    mesh=_mesh,
    out_type=jax.ShapeDtypeStruct((B * MAX_LEN, D), jnp.float32),
    scratch_types=[
        pltpu.VMEM((16,), jnp.int32),
        pltpu.VMEM((HB,), jnp.int32),
        pltpu.VMEM((HB, D), jnp.float32),
        pltpu.VMEM((HB, D), jnp.float32),
        pltpu.VMEM_SHARED((SPROWS, D), jnp.float32),
        pltpu.VMEM_SHARED((CHUNK, D), jnp.float32),
        pltpu.SemaphoreType.DMA,
        pltpu.SemaphoreType.DMA,
    ],
)
def _pos_enc_sc(len_hbm, table_hbm, tshift_hbm, pad_hbm, out_hbm, len_v,
                idx_v, buf_a, buf_b, sp_data, sp_pad, gsem, csem):
    c = lax.axis_index("c")
    s = lax.axis_index("s")
    wid = s * NC + c
    i0 = c * HALF_LEN
    row_out0 = s * MAX_LEN + i0

    lane = lax.iota(jnp.int32, 16)

    pltpu.sync_copy(len_hbm.at[pl.ds(wid * 16, 16)], len_v)
    len_b = len_v[...][0]

    bufs = (buf_a, buf_b)

    def chunk_ops(k, sp_base, fire):
        local_i = k * CHUNK
        base_i = i0 + local_i
        row_out = row_out0 + local_i
        is_data = base_i + CHUNK <= len_b
        is_pad = base_i >= len_b

        @pl.when(is_data)
        def _():
            cp = pltpu.make_async_copy(
                sp_data.at[pl.ds(local_i - sp_base, CHUNK)],
                out_hbm.at[pl.ds(row_out, CHUNK)], csem)
            cp.start() if fire else cp.wait()

        @pl.when(jnp.logical_and(~is_data, is_pad))
        def _():
            cp = pltpu.make_async_copy(sp_pad,
                                       out_hbm.at[pl.ds(row_out, CHUNK)], csem)
            cp.start() if fire else cp.wait()

        @pl.when(jnp.logical_and(~is_data, ~is_pad))
        def _():
            for h in range(CHUNK // HB):
                if fire:
                    for j in range(HB // 16):
                        rows = base_i + h * HB + j * 16 + lane
                        idx_v[pl.ds(j * 16, 16)] = jnp.where(
                            rows < len_b, rows + 1, 0)
                    pltpu.async_copy(table_hbm.at[idx_v], bufs[h],
                                     gsem).wait()
                cp = pltpu.make_async_copy(
                    bufs[h], out_hbm.at[pl.ds(row_out + h * HB, HB)], csem)
                cp.start() if fire else cp.wait()

    for p in range(NPHASE):
        sp_base = p * SPROWS

        @pl.when(s == 0)
        def _():
            pltpu.async_copy(tshift_hbm.at[pl.ds(i0 + sp_base, SPROWS)],
                             sp_data, gsem)
            if p == 0:
                pltpu.async_copy(pad_hbm, sp_pad, gsem)
                pltpu.make_async_copy(pad_hbm, sp_pad, gsem).wait()
            pltpu.make_async_copy(tshift_hbm.at[pl.ds(i0 + sp_base, SPROWS)],
                                  sp_data, gsem).wait()

        plsc.subcore_barrier()

        def fire_body(k, _, sp_base=sp_base):
            chunk_ops(k, sp_base, fire=True)
            return _

        def drain_body(k, _, sp_base=sp_base):
            chunk_ops(k, sp_base, fire=False)
            return _

        lax.fori_loop(p * PCHUNK, (p + 1) * PCHUNK, fire_body, None)
        lax.fori_loop(p * PCHUNK, (p + 1) * PCHUNK, drain_body, None)

        plsc.subcore_barrier()


def kernel(input_len, table):
    tshift = lax.slice(table, (1, 0), (MAX_LEN + 1, D))
    pad_blk = jnp.broadcast_to(table[0], (CHUNK, D))
    len_rep = jnp.repeat(input_len, 2 * 16)
    out = _pos_enc_sc(len_rep, table, tshift, pad_blk)
    return out.reshape(B, MAX_LEN, D)


# speedup vs baseline: 18.4952x; 1.0614x over previous
import functools

import jax
import jax.numpy as jnp
from jax import lax
from jax.experimental import pallas as pl
from jax.experimental.pallas import tpu as pltpu
from jax.experimental.pallas import tpu_sc as plsc

B = 16
MAX_LEN = 2048
D = 1024

NC = 2
NS = 16
HALF_LEN = MAX_LEN // NC
CHUNK = 64
NCHUNK = HALF_LEN // CHUNK
HB = 32
SPROWS = 896
PHASE_CHUNKS = (range(0, SPROWS // CHUNK), range(SPROWS // CHUNK, NCHUNK))

_mesh = plsc.VectorSubcoreMesh(core_axis_name="c", subcore_axis_name="s")


@functools.partial(
    pl.kernel,
    mesh=_mesh,
    out_type=jax.ShapeDtypeStruct((B * MAX_LEN, D), jnp.float32),
    scratch_types=[
        pltpu.VMEM((16,), jnp.int32),
        pltpu.VMEM((HB,), jnp.int32),
        pltpu.VMEM((HB,), jnp.int32),
        pltpu.VMEM((HB, D), jnp.float32),
        pltpu.VMEM((HB, D), jnp.float32),
        pltpu.VMEM_SHARED((SPROWS, D), jnp.float32),
        pltpu.VMEM_SHARED((CHUNK, D), jnp.float32),
        pltpu.SemaphoreType.DMA,
        pltpu.SemaphoreType.DMA,
        pltpu.SemaphoreType.DMA,
    ],
)
def _pos_enc_sc(len_hbm, table_hbm, tshift_hbm, pad_hbm, out_hbm, len_v,
                idx_a, idx_b, buf_a, buf_b, sp_data, sp_pad, stsem, gsem,
                csem):
    c = lax.axis_index("c")
    s = lax.axis_index("s")
    wid = s * NC + c
    i0 = c * HALF_LEN
    row_out0 = s * MAX_LEN + i0

    lane = lax.iota(jnp.int32, 16)

    pltpu.sync_copy(len_hbm.at[pl.ds(wid * 16, 16)], len_v)
    len_b = len_v[...][0]

    nd = jnp.clip(len_b - i0, 0, HALF_LEN)
    has_bnd = (nd % CHUNK) != 0
    kb = nd // CHUNK

    idxs = (idx_a, idx_b)
    bufs = (buf_a, buf_b)

    @pl.when(has_bnd)
    def _():
        for h in range(CHUNK // HB):
            for j in range(HB // 16):
                rows = i0 + kb * CHUNK + h * HB + j * 16 + lane
                idxs[h][pl.ds(j * 16, 16)] = jnp.where(rows < len_b, rows + 1,
                                                       0)
            pltpu.async_copy(table_hbm.at[idxs[h]], bufs[h], gsem)

    def chunk_ops(k, sp_base, fire):
        local_i = k * CHUNK
        row_out = row_out0 + local_i
        is_data = local_i + CHUNK <= nd
        is_pad = local_i >= nd

        @pl.when(is_data)
        def _():
            cp = pltpu.make_async_copy(
                sp_data.at[pl.ds(local_i - sp_base, CHUNK)],
                out_hbm.at[pl.ds(row_out, CHUNK)], csem)
            cp.start() if fire else cp.wait()

        @pl.when(jnp.logical_and(~is_data, is_pad))
        def _():
            cp = pltpu.make_async_copy(sp_pad,
                                       out_hbm.at[pl.ds(row_out, CHUNK)], csem)
            cp.start() if fire else cp.wait()

    for p, chunks in enumerate(PHASE_CHUNKS):
        sp_base = chunks[0] * CHUNK
        rows_p = len(chunks) * CHUNK

        @pl.when(s == 0)
        def _():
            pltpu.async_copy(tshift_hbm.at[pl.ds(i0 + sp_base, rows_p)],
                             sp_data.at[pl.ds(0, rows_p)], stsem)
            if p == 0:
                pltpu.async_copy(pad_hbm, sp_pad, stsem)
                pltpu.make_async_copy(pad_hbm, sp_pad, stsem).wait()
            pltpu.make_async_copy(tshift_hbm.at[pl.ds(i0 + sp_base, rows_p)],
                                  sp_data.at[pl.ds(0, rows_p)], stsem).wait()

        plsc.subcore_barrier()

        def fire_body(k, _, sp_base=sp_base):
            chunk_ops(k, sp_base, fire=True)
            return _

        def drain_body(k, _, sp_base=sp_base):
            chunk_ops(k, sp_base, fire=False)
            return _

        lax.fori_loop(chunks[0], chunks[-1] + 1, fire_body, None)
        lax.fori_loop(chunks[0], chunks[-1] + 1, drain_body, None)

        plsc.subcore_barrier()

    @pl.when(has_bnd)
    def _():
        row_out = row_out0 + kb * CHUNK
        for h in range(CHUNK // HB):
            pltpu.make_async_copy(table_hbm.at[idxs[h]], bufs[h],
                                  gsem).wait()
            pltpu.async_copy(bufs[h], out_hbm.at[pl.ds(row_out + h * HB, HB)],
                             csem)
        for h in range(CHUNK // HB):
            pltpu.make_async_copy(bufs[h],
                                  out_hbm.at[pl.ds(row_out + h * HB, HB)],
                                  csem).wait()


def kernel(input_len, table):
    tshift = lax.slice(table, (1, 0), (MAX_LEN + 1, D))
    pad_blk = jnp.broadcast_to(table[0], (CHUNK, D))
    len_rep = jnp.repeat(input_len, 2 * 16)
    out = _pos_enc_sc(len_rep, table, tshift, pad_blk)
    return out.reshape(B, MAX_LEN, D)
